# baseline (speedup 1.0000x reference)
import math

import jax
import jax.numpy as jnp
from jax.experimental import pallas as pl


def _pe_add_block(emb_ref, pe_ref, out_ref, *, scale, b):
    out_ref[...] = emb_ref[...] * scale + jnp.tile(pe_ref[...], (1, b))


def kernel(emb, src_org, pe):
    del src_org
    seq, b, dim = emb.shape
    scale = math.sqrt(pe.shape[-1])

    block_s = 256
    grid = (seq // block_s,)

    emb2d = emb.reshape(seq, b * dim)
    pe2d = pe[:seq, 0, :]

    out2d = pl.pallas_call(
        lambda e, p, o: _pe_add_block(e, p, o, scale=scale, b=b),
        grid=grid,
        in_specs=[
            pl.BlockSpec((block_s, b * dim), lambda i: (i, 0)),
            pl.BlockSpec((block_s, dim), lambda i: (i, 0)),
        ],
        out_specs=pl.BlockSpec((block_s, b * dim), lambda i: (i, 0)),
        out_shape=jax.ShapeDtypeStruct((seq, b * dim), emb.dtype),
    )(emb2d, pe2d)
    return out2d.reshape(seq, b, dim)


# speedup vs baseline: 3.3435x; 3.3435x over previous
import math

import jax
import jax.numpy as jnp
from jax.experimental import pallas as pl


def _pe_add_block(emb_ref, pe_ref, out_ref, *, scale):
    out_ref[...] = emb_ref[...] * scale + pe_ref[...]


def kernel(emb, src_org, pe):
    del src_org
    seq, b, dim = emb.shape
    scale = math.sqrt(pe.shape[-1])

    block_s = 256
    grid = (seq // block_s,)

    return pl.pallas_call(
        lambda e, p, o: _pe_add_block(e, p, o, scale=scale),
        grid=grid,
        in_specs=[
            pl.BlockSpec((block_s, b, dim), lambda i: (i, 0, 0)),
            pl.BlockSpec((block_s, 1, dim), lambda i: (0, 0, 0)),
        ],
        out_specs=pl.BlockSpec((block_s, b, dim), lambda i: (i, 0, 0)),
        out_shape=jax.ShapeDtypeStruct((seq, b, dim), emb.dtype),
    )(emb, pe[:seq])


# speedup vs baseline: 3.6192x; 1.0825x over previous
import math

import jax
import jax.numpy as jnp
from jax.experimental import pallas as pl


def _pe_add_block(emb_ref, pe_ref, out_ref, *, scale):
    out_ref[...] = emb_ref[...] * scale + pe_ref[...]


def kernel(emb, src_org, pe):
    del src_org
    seq, b, dim = emb.shape
    scale = math.sqrt(pe.shape[-1])

    block_s = 256
    grid = (seq // block_s,)

    return pl.pallas_call(
        lambda e, p, o: _pe_add_block(e, p, o, scale=scale),
        grid=grid,
        in_specs=[
            pl.BlockSpec((block_s, b, dim), lambda i: (i, 0, 0)),
            pl.BlockSpec((block_s, 1, dim), lambda i: (i, 0, 0)),
        ],
        out_specs=pl.BlockSpec((block_s, b, dim), lambda i: (i, 0, 0)),
        out_shape=jax.ShapeDtypeStruct((seq, b, dim), emb.dtype),
    )(emb, pe)
